# Initial kernel scaffold; baseline (speedup 1.0000x reference)
import jax
import jax.numpy as jnp
from jax.experimental import pallas as pl


def kernel(adjacency, params):
    raise NotImplementedError("write your pallas kernel here")


# speedup vs baseline: 14.0439x; 14.0439x over previous
import jax
import jax.numpy as jnp
from jax.experimental import pallas as pl
from jax.experimental.pallas import tpu as pltpu

_EPS = 1e-5
_V = 64
_B = 64
_E = 64

_BF = jnp.bfloat16


def _dt(a, b):
    return jax.lax.dot_general(a.astype(_BF), b.astype(_BF),
                               (((1,), (1,)), ((), ())),
                               preferred_element_type=jnp.float32)


def _mm(a, b):
    return jax.lax.dot_general(a.astype(_BF), b.astype(_BF),
                               (((1,), (0,)), ((), ())),
                               preferred_element_type=jnp.float32)


def _ln(x, w, b):
    m = jnp.mean(x, axis=-1, keepdims=True)
    v = jnp.mean((x - m) ** 2, axis=-1, keepdims=True)
    return (x - m) / jnp.sqrt(v + _EPS) * w + b


def _mlp3(x, W1, b1, W2, b2, W3, b3):
    h = jax.nn.relu(_dt(x, W1) + b1)
    h = jax.nn.relu(_dt(h, W2) + b2)
    return _dt(h, W3) + b3


def _stage_a(wm_ref, vr_ref, e1_ref, c2_ref):
    nemb = wm_ref[0]
    Ws, Wr, We, Wg = wm_ref[1], wm_ref[2], wm_ref[3], wm_ref[4]
    W2, W3 = wm_ref[5], wm_ref[6]
    vr = vr_ref[...]
    edge0, g0 = vr[0:1], vr[1:2]
    b1, b2, b3 = vr[2:3], vr[3:4], vr[4:5]
    lnw, lnb = vr[5:6], vr[6:7]

    c0 = _dt(edge0, We) + _dt(g0, Wg) + b1
    P = _dt(nemb, Ws)
    Q = _dt(nemb, Wr)
    H = jax.nn.relu(P[:, None, :] + Q[None, :, :] + c0[None, :, :])
    H = H.reshape(_V * _V, _E)
    H = jax.nn.relu(_dt(H, W2) + b2)
    E1 = _dt(H, W3) + b3
    e1_ref[...] = E1
    E1n = _ln(edge0 + E1, lnw, lnb)
    c2_ref[...] = _dt(E1n, We)


def _stage_b(adj_ref, et_ref, wm_ref, vr_ref, wp_ref, bp_ref, out_ref, stop_ref):
    adj = adj_ref[0].astype(jnp.float32)
    E1 = et_ref[0]
    C2 = et_ref[1]

    nemb = wm_ref[0]
    Ws, Wr, Wge = wm_ref[1], wm_ref[2], wm_ref[3]
    We2, We3 = wm_ref[4], wm_ref[5]
    Wnn, Wns, Wnr, Wng = wm_ref[6], wm_ref[7], wm_ref[8], wm_ref[9]
    Wn2, Wn3 = wm_ref[10], wm_ref[11]
    Wgn, Wge2, Wgg = wm_ref[12], wm_ref[13], wm_ref[14]
    Wg2, Wg3 = wm_ref[15], wm_ref[16]
    Wq, Wk, Wv, Wo = wm_ref[17], wm_ref[18], wm_ref[19], wm_ref[20]
    S1, S2, S3 = wm_ref[21], wm_ref[22], wm_ref[23]
    R1, R2, R3 = wm_ref[24], wm_ref[25], wm_ref[26]
    T1, T2 = wm_ref[27], wm_ref[28]
    vr = vr_ref[...]
    g0 = vr[0:1]
    be1, be2, be3 = vr[1:2], vr[2:3], vr[3:4]
    bn1, bn2, bn3 = vr[4:5], vr[5:6], vr[6:7]
    bg1, bg2, bg3 = vr[7:8], vr[8:9], vr[9:10]
    lnn_w, lnn_b = vr[10:11], vr[11:12]
    lng_w, lng_b = vr[12:13], vr[13:14]
    lpn_w, lpn_b = vr[14:15], vr[15:16]
    lpg_w, lpg_b = vr[16:17], vr[17:18]
    bs1, bs2, bs3 = vr[18:19], vr[19:20], vr[20:21]
    br1, br2, br3 = vr[21:22], vr[22:23], vr[23:24]
    bt1, bt2 = vr[24:25], vr[25:26]
    T3 = vr[26:27]
    bt3 = vr[27, 0]
    lnorm = vr[27, 1]
    Wp = wp_ref[...]
    bp = bp_ref[...]

    T = adj[:, :, None] * E1.reshape(_V, _V, _E)
    sagg = jnp.sum(T, axis=1)
    ragg = jnp.sum(T, axis=0)
    eagg = jnp.sum(sagg, axis=0, keepdims=True)

    h = jax.nn.relu(_dt(nemb, Wnn) + _dt(sagg, Wns) + _dt(ragg, Wnr)
                    + _dt(g0, Wng) + bn1)
    h = jax.nn.relu(_dt(h, Wn2) + bn2)
    nn1 = _dt(h, Wn3) + bn3
    nagg = jnp.sum(nn1, axis=0, keepdims=True)

    gh = jax.nn.relu(_dt(nagg, Wgn) + _dt(eagg, Wge2) + _dt(g0, Wgg) + bg1)
    gh = jax.nn.relu(_dt(gh, Wg2) + bg2)
    ng1 = _dt(gh, Wg3) + bg3

    nodes1 = _ln(nemb + nn1, lnn_w, lnn_b)
    g1 = _ln(g0 + ng1, lng_w, lng_b)

    A2 = _dt(nodes1, Ws)
    B2 = _dt(nodes1, Wr)
    gterm = _dt(g1, Wge) + be1
    H = jax.nn.relu(C2.reshape(_V, _V, _E) + A2[:, None, :] + B2[None, :, :]
                    + gterm[None, :, :])
    H = H.reshape(_V * _V, _E)
    H = jax.nn.relu(_dt(H, We2) + be2)
    E2 = _dt(H, We3) + be3

    T2v = adj[:, :, None] * E2.reshape(_V, _V, _E)
    sagg2 = jnp.sum(T2v, axis=1)
    ragg2 = jnp.sum(T2v, axis=0)
    eagg2 = jnp.sum(sagg2, axis=0, keepdims=True)

    h = jax.nn.relu(_dt(nodes1, Wnn) + _dt(sagg2, Wns) + _dt(ragg2, Wnr)
                    + _dt(g1, Wng) + bn1)
    h = jax.nn.relu(_dt(h, Wn2) + bn2)
    nn2 = _dt(h, Wn3) + bn3
    nagg2 = jnp.sum(nn2, axis=0, keepdims=True)

    gh = jax.nn.relu(_dt(nagg2, Wgn) + _dt(eagg2, Wge2) + _dt(g1, Wgg) + bg1)
    gh = jax.nn.relu(_dt(gh, Wg2) + bg2)
    ng2 = _dt(gh, Wg3) + bg3

    nodes2 = _ln(nodes1 + nn2, lnn_w, lnn_b)
    g2 = _ln(g1 + ng2, lng_w, lng_b)

    nf0 = _dt(nodes2, Wp) + bp
    q, k, v = nf0[:, :_E], nf0[:, _E:2 * _E], nf0[:, 2 * _E:]
    qp, kp, vp = _dt(q, Wq), _dt(k, Wk), _dt(v, Wv)
    parts = []
    for hh in range(4):
        sl = slice(16 * hh, 16 * (hh + 1))
        lg = _dt(qp[:, sl], kp[:, sl]) * 0.25
        lg = lg - jnp.max(lg, axis=1, keepdims=True)
        w = jnp.exp(lg)
        w = w / jnp.sum(w, axis=1, keepdims=True)
        parts.append(_mm(w, vp[:, sl]))
    o = jnp.concatenate(parts, axis=1)
    nf = _ln(_dt(o, Wo), lpn_w, lpn_b)
    gf = _ln(g2, lpg_w, lpg_b)

    s = _mlp3(nf, S1, bs1, S2, bs2, S3, bs3)
    r = _mlp3(nf, R1, br1, R2, br2, R3, br3)
    logits = _dt(s, r) / lnorm
    out_ref[...] = logits[None]

    th = jax.nn.relu(_dt(gf, T1) + bt1)
    th = jax.nn.relu(_dt(th, T2) + bt2)
    stop = (jnp.sum(th.astype(_BF).astype(jnp.float32)
                    * T3.astype(_BF).astype(jnp.float32)) + bt3) / lnorm
    stop_ref[...] = jnp.broadcast_to(jnp.reshape(stop, (1, 1, 1)), (1, 1, 128))


def kernel(adjacency, params):
    p = params
    (W1e, be1), (We2, be2), (We3, be3) = p['edge_mlp']
    (W1n, bn1), (Wn2, bn2), (Wn3, bn3) = p['node_mlp']
    (W1g, bg1), (Wg2, bg2), (Wg3, bg3) = p['global_mlp']
    nemb = p['node_emb']
    edge0 = p['edge_emb'].reshape(_E)
    g0 = p['global_emb'].reshape(_E)
    lne_w, lne_b = p['ln_edges']
    lnn_w, lnn_b = p['ln_nodes']
    lng_w, lng_b = p['ln_globals']
    lpn_w, lpn_b = p['ln_post_nodes']
    lpg_w, lpg_b = p['ln_post_globals']
    Wp, bp = p['attn_proj']
    at = p['attn']
    (S1, bs1), (S2, bs2), (S3, bs3) = p['senders_mlp']
    (R1, br1), (R2, br2), (R3, br3) = p['receivers_mlp']
    (T1, bt1), (T2, bt2), (T3, bt3) = p['stop_mlp']
    lnorm = p['logits_norm']

    wmA = jnp.stack([nemb, W1e[:, 64:128], W1e[:, 128:192], W1e[:, 0:64],
                     W1e[:, 192:256], We2, We3])
    vrA = jnp.stack([edge0, g0, be1, be2, be3, lne_w, lne_b])

    E1, C2 = pl.pallas_call(
        _stage_a,
        out_shape=(jax.ShapeDtypeStruct((_V * _V, _E), jnp.float32),
                   jax.ShapeDtypeStruct((_V * _V, _E), jnp.float32)),
    )(wmA, vrA)
    etab = jnp.stack([E1, C2])

    wmB = jnp.stack([
        nemb, W1e[:, 64:128], W1e[:, 128:192], W1e[:, 192:256], We2, We3,
        W1n[:, 0:64], W1n[:, 64:128], W1n[:, 128:192], W1n[:, 192:256],
        Wn2, Wn3,
        W1g[:, 0:64], W1g[:, 64:128], W1g[:, 128:192], Wg2, Wg3,
        at['Wq'], at['Wk'], at['Wv'], at['Wo'],
        S1, S2, S3, R1, R2, R3, T1, T2,
    ])
    scal = jnp.zeros((_E,), jnp.float32).at[0].set(bt3[0]).at[1].set(lnorm[0])
    vrB = jnp.stack([
        g0, be1, be2, be3, bn1, bn2, bn3, bg1, bg2, bg3,
        lnn_w, lnn_b, lng_w, lng_b, lpn_w, lpn_b, lpg_w, lpg_b,
        bs1, bs2, bs3, br1, br2, br3, bt1, bt2, T3.reshape(_E), scal,
    ])

    pair_logits, stop_col = pl.pallas_call(
        _stage_b,
        grid=(_B,),
        in_specs=[
            pl.BlockSpec((1, _V, _V), lambda b: (b, 0, 0)),
            pl.BlockSpec((2, _V * _V, _E), lambda b: (0, 0, 0)),
            pl.BlockSpec((29, _E, _E), lambda b: (0, 0, 0)),
            pl.BlockSpec((28, _E), lambda b: (0, 0)),
            pl.BlockSpec((3 * _E, _E), lambda b: (0, 0)),
            pl.BlockSpec((1, 3 * _E), lambda b: (0, 0)),
        ],
        out_specs=[
            pl.BlockSpec((1, _V, _V), lambda b: (b, 0, 0)),
            pl.BlockSpec((1, 1, 128), lambda b: (b, 0, 0)),
        ],
        out_shape=(jax.ShapeDtypeStruct((_B, _V, _V), jnp.float32),
                   jax.ShapeDtypeStruct((_B, 1, 128), jnp.float32)),
        compiler_params=pltpu.CompilerParams(
            dimension_semantics=("arbitrary",)),
    )(adjacency, etab, wmB, vrB, Wp, bp.reshape(1, 3 * _E))

    fwd = jnp.concatenate([pair_logits.reshape(_B, _V * _V),
                           stop_col[:, 0, :1]], axis=1)
    bwd = jnp.zeros((_B, _V * _V + 1), jnp.float32)
    return fwd, bwd
